# baseline (speedup 1.0000x reference)
import functools

import jax
import jax.numpy as jnp
from jax import lax
from jax.experimental import pallas as pl
from jax.experimental.pallas import tpu as pltpu
from jax.experimental.pallas import tpu_sc as plsc

H = 64
NUM_BITS = 20
BATCH = 16384

_info = plsc.get_sparse_core_info()
_NC, _NS, _L = _info.num_cores, _info.num_subcores, _info.num_lanes
_NW = _NC * _NS
_BPW = BATCH // _NW
_CHUNK = 128
_NCHUNK = _BPW // _CHUNK


def _pack_bits_body(states_ref, idx_ref):
    powers = (1 << lax.broadcasted_iota(jnp.int32, (1, NUM_BITS), 1))
    idx_ref[...] = jnp.sum(states_ref[...] * powers, axis=1)


def _gather_body(idx_hbm, emb_hbm, out_hbm, idx_v, rows_v, sem):
    wid = lax.axis_index("s") * _NC + lax.axis_index("c")

    pltpu.sync_copy(idx_hbm.at[pl.ds(wid * _NCHUNK, _NCHUNK)], idx_v)

    copies = [
        pltpu.async_copy(
            emb_hbm.at[idx_v.at[c]],
            rows_v.at[pl.ds(c * _CHUNK, _CHUNK)],
            sem,
        )
        for c in range(_NCHUNK)
    ]
    for cp in copies:
        cp.wait()

    pltpu.sync_copy(rows_v, out_hbm.at[pl.ds(wid * _BPW, _BPW)])


@jax.jit
def kernel(states, emb):
    idx = pl.pallas_call(
        _pack_bits_body,
        out_shape=jax.ShapeDtypeStruct((BATCH,), jnp.int32),
    )(states)

    mesh = plsc.VectorSubcoreMesh(core_axis_name="c", subcore_axis_name="s")
    gather = functools.partial(
        pl.kernel,
        mesh=mesh,
        out_type=jax.ShapeDtypeStruct((BATCH, H), jnp.float32),
        scratch_types=[
            pltpu.VMEM((_NCHUNK, _CHUNK), jnp.int32),
            pltpu.VMEM((_BPW, H), jnp.float32),
            pltpu.SemaphoreType.DMA,
        ],
        compiler_params=pltpu.CompilerParams(use_tc_tiling_on_sc=False),
    )(_gather_body)
    return gather(idx.reshape(BATCH // _CHUNK, _CHUNK), emb)


# speedup vs baseline: 1.6081x; 1.6081x over previous
import functools

import jax
import jax.numpy as jnp
from jax import lax
from jax.experimental import pallas as pl
from jax.experimental.pallas import tpu as pltpu
from jax.experimental.pallas import tpu_sc as plsc

H = 64
NUM_BITS = 20
BATCH = 16384

_info = plsc.get_sparse_core_info()
_NC, _NS, _L = _info.num_cores, _info.num_subcores, _info.num_lanes
_NW = _NC * _NS
_BPW = BATCH // _NW
_NG = _BPW // _L


def _pack_bits_body(states_ref, idx_ref):
    powers = (1 << lax.broadcasted_iota(jnp.int32, (1, NUM_BITS), 1))
    idx_ref[...] = jnp.sum(states_ref[...] * powers, axis=1)


def _gather_body(idx_hbm, emb_hbm, out_hbm, idx_v, rows_v, sem):
    wid = lax.axis_index("s") * _NC + lax.axis_index("c")
    base = wid * _BPW

    pltpu.sync_copy(idx_hbm.at[pl.ds(base, _BPW)], idx_v)

    def fire_body(g, _):
        rows16 = idx_v[pl.ds(g * _L, _L)]
        for i in range(_L):
            pltpu.async_copy(
                emb_hbm.at[rows16[i]], rows_v.at[g * _L + i], sem)
        return _

    lax.fori_loop(0, _NG, fire_body, None)

    for g in range(_NG):
        pltpu.make_async_copy(
            emb_hbm.at[pl.ds(0, _L)],
            rows_v.at[pl.ds(g * _L, _L)],
            sem,
        ).wait()

    pltpu.sync_copy(rows_v, out_hbm.at[pl.ds(base, _BPW)])


@jax.jit
def kernel(states, emb):
    idx = pl.pallas_call(
        _pack_bits_body,
        out_shape=jax.ShapeDtypeStruct((BATCH,), jnp.int32),
    )(states)

    mesh = plsc.VectorSubcoreMesh(core_axis_name="c", subcore_axis_name="s")
    gather = functools.partial(
        pl.kernel,
        mesh=mesh,
        out_type=jax.ShapeDtypeStruct((BATCH, H), jnp.float32),
        scratch_types=[
            pltpu.VMEM((_BPW,), jnp.int32),
            pltpu.VMEM((_BPW, H), jnp.float32),
            pltpu.SemaphoreType.DMA,
        ],
    )(_gather_body)
    return gather(idx, emb)


# speedup vs baseline: 9.4898x; 5.9014x over previous
import functools

import jax
import jax.numpy as jnp
from jax import lax
from jax.experimental import pallas as pl
from jax.experimental.pallas import tpu as pltpu
from jax.experimental.pallas import tpu_sc as plsc

H = 64
NUM_BITS = 20
BATCH = 16384
V = 2**NUM_BITS

_info = plsc.get_sparse_core_info()
_NC, _NS, _L = _info.num_cores, _info.num_subcores, _info.num_lanes
_NW = _NC * _NS
_BPW = BATCH // _NW
_NG = _BPW // _L
_STREAM = 128
_NSTREAM = _BPW * H // _STREAM // H
_TILE_WORDS = V // 128 * 1024


def _gather_body(states_hbm, emb_hbm, out_hbm, st_v, base_v, idx_v, col_v,
                 sem):
    wid = lax.axis_index("s") * _NC + lax.axis_index("c")
    base = wid * _BPW

    pltpu.sync_copy(states_hbm.at[:, pl.ds(base, _BPW)], st_v)

    def pack_body(g, carry):
        acc = st_v[0, pl.ds(g * _L, _L)]
        for j in range(1, NUM_BITS):
            acc = acc + (st_v[j, pl.ds(g * _L, _L)] << j)
        base_v[pl.ds(g * _L, _L)] = (acc >> 7) * 1024 + (acc & 127)
        return carry

    lax.fori_loop(0, _NG, pack_body, None)

    def idx_body(g, carry):
        b16 = base_v[pl.ds(g * _L, _L)]
        for h in range(H):
            off = (h // 8) * _TILE_WORDS + (h % 8) * 128
            idx_v[pl.ds(h * _BPW + g * _L, _L)] = b16 + off
        return carry

    lax.fori_loop(0, _NG, idx_body, None)

    def fire_body(k, carry):
        pltpu.async_copy(
            emb_hbm.at[idx_v.at[pl.ds(k * _STREAM, _STREAM)]],
            col_v.at[k // _NSTREAM, pl.ds((k % _NSTREAM) * _STREAM, _STREAM)],
            sem,
        )
        return carry

    lax.fori_loop(0, H * _NSTREAM, fire_body, None)

    for q in range(H):
        pltpu.make_async_copy(
            emb_hbm.at[pl.ds(0, _BPW)],
            col_v.at[q],
            sem,
        ).wait()

    pltpu.sync_copy(col_v, out_hbm.at[:, pl.ds(base, _BPW)])


@jax.jit
def kernel(states, emb):
    emb_flat = emb.reshape(V // 128, 128, 8, H // 8)
    emb_flat = emb_flat.transpose(2, 0, 3, 1).reshape(-1)
    mesh = plsc.VectorSubcoreMesh(core_axis_name="c", subcore_axis_name="s")
    gather = functools.partial(
        pl.kernel,
        mesh=mesh,
        out_type=jax.ShapeDtypeStruct((H, BATCH), jnp.float32),
        scratch_types=[
            pltpu.VMEM((NUM_BITS, _BPW), jnp.int32),
            pltpu.VMEM((_BPW,), jnp.int32),
            pltpu.VMEM((H * _BPW,), jnp.int32),
            pltpu.VMEM((H, _BPW), jnp.float32),
            pltpu.SemaphoreType.DMA,
        ],
        compiler_params=pltpu.CompilerParams(use_tc_tiling_on_sc=False),
    )(_gather_body)
    out_t = gather(states.T, emb_flat)
    return out_t.T


# speedup vs baseline: 10.0227x; 1.0562x over previous
import functools

import jax
import jax.numpy as jnp
from jax import lax
from jax.experimental import pallas as pl
from jax.experimental.pallas import tpu as pltpu
from jax.experimental.pallas import tpu_sc as plsc

H = 64
NUM_BITS = 20
BATCH = 16384
V = 2**NUM_BITS

_info = plsc.get_sparse_core_info()
_NC, _NS, _L = _info.num_cores, _info.num_subcores, _info.num_lanes
_NW = _NC * _NS
_BPW = BATCH // _NW
_NG = _BPW // _L
_STREAM = 128
_NCH = _BPW // _STREAM
_TILE_WORDS = V // 128 * 1024
_NB = H // 8


def _gather_body(states_hbm, emb_hbm, out_hbm, st_v, base_v, idx_v, col_v,
                 sem_out, *sems):
    wid = lax.axis_index("s") * _NC + lax.axis_index("c")
    base = wid * _BPW

    pltpu.sync_copy(states_hbm.at[:, pl.ds(base, _BPW)], st_v)

    def pack_body(g, carry):
        acc = st_v[0, pl.ds(g * _L, _L)]
        for j in range(1, NUM_BITS):
            acc = acc + (st_v[j, pl.ds(g * _L, _L)] << j)
        base_v[pl.ds(g * _L, _L)] = (acc >> 7) * 1024 + (acc & 127)
        return carry

    lax.fori_loop(0, _NG, pack_body, None)

    def make_idx_body(a):
        def idx_body(g, carry):
            b16 = base_v[pl.ds(g * _L, _L)]
            c = g // (_STREAM // _L)
            o = (g % (_STREAM // _L)) * _L
            for s in range(8):
                pos = ((a * _NCH + c) * 8 + s) * _STREAM + o
                idx_v[pl.ds(pos, _L)] = b16 + (a * _TILE_WORDS + s * 128)
            return carry
        return idx_body

    pending = []
    out_writes = []

    def finish_band(a, copies):
        for cp in copies:
            cp.wait()
        out_writes.append(pltpu.async_copy(
            col_v.at[a], out_hbm.at[a, pl.ds(wid * _NCH, _NCH)], sem_out))

    prev = None
    for a in range(_NB):
        lax.fori_loop(0, _NG, make_idx_body(a), None)
        copies = []
        for c in range(_NCH):
            for s in range(8):
                k = (a * _NCH + c) * 8 + s
                copies.append(pltpu.async_copy(
                    emb_hbm.at[idx_v.at[pl.ds(k * _STREAM, _STREAM)]],
                    col_v.at[a, c, s],
                    sems[a],
                ))
        if prev is not None:
            finish_band(*prev)
        prev = (a, copies)
    finish_band(*prev)

    for wr in out_writes:
        wr.wait()


@jax.jit
def kernel(states, emb):
    emb_flat = emb.reshape(V // 128, 128, 8, H // 8)
    emb_flat = emb_flat.transpose(2, 0, 3, 1).reshape(-1)
    mesh = plsc.VectorSubcoreMesh(core_axis_name="c", subcore_axis_name="s")
    gather = functools.partial(
        pl.kernel,
        mesh=mesh,
        out_type=jax.ShapeDtypeStruct((_NB, BATCH // _STREAM, 8, _STREAM),
                                      jnp.float32),
        scratch_types=[
            pltpu.VMEM((NUM_BITS, _BPW), jnp.int32),
            pltpu.VMEM((_BPW,), jnp.int32),
            pltpu.VMEM((H * _BPW,), jnp.int32),
            pltpu.VMEM((_NB, _NCH, 8, _STREAM), jnp.float32),
            pltpu.SemaphoreType.DMA,
        ] + [pltpu.SemaphoreType.DMA] * _NB,
        compiler_params=pltpu.CompilerParams(use_tc_tiling_on_sc=False),
    )(_gather_body)
    out4 = gather(states.T, emb_flat)
    return out4.transpose(1, 3, 0, 2).reshape(BATCH, H)


# speedup vs baseline: 10.2312x; 1.0208x over previous
import functools

import jax
import jax.numpy as jnp
from jax import lax
from jax.experimental import pallas as pl
from jax.experimental.pallas import tpu as pltpu
from jax.experimental.pallas import tpu_sc as plsc

H = 64
NUM_BITS = 20
BATCH = 16384
V = 2**NUM_BITS

_info = plsc.get_sparse_core_info()
_NC, _NS, _L = _info.num_cores, _info.num_subcores, _info.num_lanes
_NW = _NC * _NS
_BPW = BATCH // _NW
_NG = _BPW // _L
_STREAM = 512
_NCH = _BPW // _STREAM
_TILE_WORDS = V // 128 * 1024
_NB = H // 8


def _gather_body(states_hbm, emb_hbm, out_hbm, st_v, base_v, idx_v, col_v,
                 sem_out, *sems):
    wid = lax.axis_index("s") * _NC + lax.axis_index("c")
    base = wid * _BPW

    pltpu.sync_copy(states_hbm.at[:, pl.ds(base, _BPW)], st_v)

    def pack_body(g, carry):
        acc = st_v[0, pl.ds(g * _L, _L)]
        for j in range(1, NUM_BITS):
            acc = acc + (st_v[j, pl.ds(g * _L, _L)] << j)
        base_v[pl.ds(g * _L, _L)] = (acc >> 7) * 1024 + (acc & 127)
        return carry

    lax.fori_loop(0, _NG, pack_body, None)

    def make_idx_body(a):
        def idx_body(g, carry):
            tbl = g // 8
            vg = g % 8
            b16 = base_v[pl.ds(tbl * 128 + vg * _L, _L)]
            for s in range(8):
                pos = a * 4096 + tbl * 1024 + s * 128 + vg * _L
                idx_v[pl.ds(pos, _L)] = b16 + (a * _TILE_WORDS + s * 128)
            return carry
        return idx_body

    pending = []
    out_writes = []

    def finish_band(a, copies):
        for cp in copies:
            cp.wait()
        out_writes.append(pltpu.async_copy(
            col_v.at[pl.ds(a * 4096, 4096)],
            out_hbm.at[pl.ds((a * (BATCH // 128) + 4 * wid) * 1024, 4096)],
            sem_out))

    prev = None
    for a in range(_NB):
        lax.fori_loop(0, _NG, make_idx_body(a), None)
        copies = []
        for tbl in range(4):
            for q in range(2):
                pos = a * 4096 + tbl * 1024 + q * _STREAM
                copies.append(pltpu.async_copy(
                    emb_hbm.at[idx_v.at[pl.ds(pos, _STREAM)]],
                    col_v.at[pl.ds(pos, _STREAM)],
                    sems[a],
                ))
        if prev is not None:
            finish_band(*prev)
        prev = (a, copies)
    finish_band(*prev)

    for wr in out_writes:
        wr.wait()


@jax.jit
def kernel(states, emb):
    emb_flat = emb.reshape(V // 128, 128, 8, H // 8)
    emb_flat = emb_flat.transpose(2, 0, 3, 1).reshape(-1)
    mesh = plsc.VectorSubcoreMesh(core_axis_name="c", subcore_axis_name="s")
    gather = functools.partial(
        pl.kernel,
        mesh=mesh,
        out_type=jax.ShapeDtypeStruct((BATCH * H,), jnp.float32),
        scratch_types=[
            pltpu.VMEM((NUM_BITS, _BPW), jnp.int32),
            pltpu.VMEM((_BPW,), jnp.int32),
            pltpu.VMEM((H * _BPW,), jnp.int32),
            pltpu.VMEM((H * _BPW,), jnp.float32),
            pltpu.SemaphoreType.DMA,
        ] + [pltpu.SemaphoreType.DMA] * _NB,
        compiler_params=pltpu.CompilerParams(use_tc_tiling_on_sc=False),
    )(_gather_body)
    out4 = gather(states.T, emb_flat).reshape(_NB, BATCH // 128, 8, 128)
    return out4.transpose(1, 3, 0, 2).reshape(BATCH, H)
